# baseline (speedup 1.0000x reference)
import jax
import jax.numpy as jnp
from jax import lax
from jax.experimental import pallas as pl
from jax.experimental.pallas import tpu as pltpu
from jax.experimental.pallas import tpu_sc as plsc

_FACTOR = 64
_BATCH = 16384
_ROWS = 1000000
_NC = 2
_NS = 16
_L = 16
_NW = _NC * _NS
_BPW = _BATCH // _NW
_CW = 512
_NCHUNK = (_ROWS + _CW - 1) // _CW
_KMAX = (_NCHUNK + _NW - 1) // _NW


def _iota16():
    return lax.iota(jnp.int32, _L)


def _gather_body(wu_hbm, wi_hbm, tu_hbm, ti_hbm, user_hbm, item_hbm,
                 stg_u, stg_i,
                 idxbuf, lb, lr, chunk0, chunk1, rowbuf, bidx, sem0, sem1):
    wid = lax.axis_index("s") * _NC + lax.axis_index("c")
    lane = _iota16()
    neg1 = jnp.full((_L,), -1, jnp.int32)

    for wt_hbm, tail_hbm, ix_hbm, stg in ((wu_hbm, tu_hbm, user_hbm, stg_u),
                                          (wi_hbm, ti_hbm, item_hbm, stg_i)):
        def scan_blk(blk, cnt):
            pltpu.sync_copy(ix_hbm.at[pl.ds(blk * 16, 16)], idxbuf)

            def scan4(v4, cnt):
                for u in range(4):
                    v = v4 * 4 + u
                    iv = idxbuf[v >> 3, pl.ds((v & 7) * _L, _L)]
                    c = lax.shift_right_logical(iv, 9)
                    own = (c & (_NW - 1)) == wid
                    csum = plsc.cumsum(jnp.where(own, 1, 0))
                    pos = cnt + csum - 1
                    bvec = blk * 2048 + v * _L + lane
                    plsc.store_scatter(lb, [pos], bvec, mask=own)
                    plsc.store_scatter(lr, [pos], iv, mask=own)
                    cnt = cnt + csum[_L - 1]
                return cnt

            return lax.fori_loop(0, 32, scan4, cnt)

        cnt = lax.fori_loop(0, 8, scan_blk, jnp.int32(0))
        nv = lax.shift_right_logical(cnt + _L - 1, 4)

        def clr(v, _):
            bidx[pl.ds(v * _L, _L)] = neg1
            return 0

        lax.fori_loop(0, 8, clr, 0)

        def fire(c, buf, sem):
            @pl.when(c < _NCHUNK - 1)
            def _():
                pltpu.async_copy(
                    wt_hbm.at[:, pl.ds(pl.multiple_of(c * _CW, _CW), _CW)],
                    buf, sem)

            @pl.when(c == _NCHUNK - 1)
            def _():
                pltpu.async_copy(tail_hbm, buf.at[:, pl.ds(0, 128)], sem)

        def wait(c, buf, sem):
            @pl.when(c < _NCHUNK - 1)
            def _():
                pltpu.make_async_copy(
                    wt_hbm.at[:, pl.ds(0, _CW)], buf, sem).wait()

            @pl.when(c == _NCHUNK - 1)
            def _():
                pltpu.make_async_copy(
                    tail_hbm, buf.at[:, pl.ds(0, 128)], sem).wait()

        def flush(mv):
            pltpu.sync_copy(rowbuf,
                            stg.at[plsc.Indices(bidx, ignored_value=-1)])
            lax.fori_loop(0, 8, clr, 0)

        def process(c, buf):
            base_r = c * _CW

            def pvec(v, mv):
                rv = lr[pl.ds(v * _L, _L)]
                bv = lb[pl.ds(v * _L, _L)]
                inr = (v * _L + lane) < cnt
                sel = (lax.shift_right_logical(rv, 9) == c) & inr
                npos = plsc.all_reduce_population_count(sel)
                hit = npos[0] > 0

                @pl.when(hit)
                def _():
                    x = (rv - base_r) & (_CW - 1)
                    row0 = (mv & 7) * _L
                    rowv = row0 + lane
                    fvec = jnp.zeros((_L,), jnp.int32)
                    for _f in range(_FACTOR):
                        vals = plsc.load_gather(buf, [fvec, x])
                        plsc.store_scatter(rowbuf, [rowv, fvec], vals)
                        fvec = fvec + 1
                    bidx[pl.ds(row0, _L)] = jnp.where(sel, bv, -1)

                mv2 = jnp.where(hit, mv + 1, mv)

                @pl.when(hit & ((mv2 & 7) == 0))
                def _():
                    flush(mv2)

                return mv2

            mv = lax.fori_loop(0, nv, pvec, jnp.int32(0))

            @pl.when((mv & 7) != 0)
            def _():
                flush(mv)

        bufs = (chunk0, chunk1)
        sems = (sem0, sem1)
        fire(wid, chunk0, sem0)
        lim = jnp.int32(_NCHUNK)

        def pair(k, _):
            for par in range(2):
                s = k * 2 + par
                c = wid + s * _NW
                cn = wid + (s + 1) * _NW

                @pl.when(cn < lim)
                def _():
                    fire(cn, bufs[1 - par], sems[1 - par])

                @pl.when(c < lim)
                def _():
                    wait(c, bufs[par], sems[par])
                    process(c, bufs[par])
            return 0

        lax.fori_loop(0, _KMAX // 2, pair, 0)


def _dot_body(stg_u, stg_i, out_hbm, su, si, out_v, sem):
    wid = lax.axis_index("s") * _NC + lax.axis_index("c")
    lane = _iota16()

    for half in range(2):
        row0 = wid * _BPW + half * 256
        pltpu.sync_copy(stg_u.at[pl.ds(row0, 256)], su)
        pltpu.sync_copy(stg_i.at[pl.ds(row0, 256)], si)

        def group(g, _):
            bvec = g * _L + lane
            acc = jnp.zeros((_L,), jnp.float32)
            fvec = jnp.zeros((_L,), jnp.int32)
            for _f in range(_FACTOR):
                u = plsc.load_gather(su, [bvec, fvec])
                v = plsc.load_gather(si, [bvec, fvec])
                acc = acc + u * v
                fvec = fvec + 1
            out_v[pl.ds(half * 256 + g * _L, _L)] = acc
            return 0

        lax.fori_loop(0, 256 // _L, group, 0)

    pltpu.sync_copy(out_v, out_hbm.at[pl.ds(wid * _BPW, _BPW)])


def kernel(user, item, W_user, W_item):
    user = user.astype(jnp.int32).reshape(128, 128)
    item = item.astype(jnp.int32).reshape(128, 128)
    mesh = plsc.VectorSubcoreMesh(core_axis_name="c", subcore_axis_name="s")
    params = pltpu.CompilerParams(needs_layout_passes=False)

    gather = pl.kernel(
        _gather_body,
        out_type=(
            jax.ShapeDtypeStruct((_BATCH, 128), jnp.float32),
            jax.ShapeDtypeStruct((_BATCH, 128), jnp.float32),
        ),
        mesh=mesh,
        compiler_params=params,
        scratch_types=[
            pltpu.VMEM((16, 128), jnp.int32),
            pltpu.VMEM((_BATCH,), jnp.int32),
            pltpu.VMEM((_BATCH,), jnp.int32),
            pltpu.VMEM((_FACTOR, _CW), jnp.float32),
            pltpu.VMEM((_FACTOR, _CW), jnp.float32),
            pltpu.VMEM((128, 128), jnp.float32),
            pltpu.VMEM((128,), jnp.int32),
            pltpu.SemaphoreType.DMA,
            pltpu.SemaphoreType.DMA,
        ],
    )
    ntail = _ROWS - (_NCHUNK - 1) * _CW
    tail_u = jnp.pad(W_user[_ROWS - ntail:].T, ((0, 0), (0, 128 - ntail)))
    tail_i = jnp.pad(W_item[_ROWS - ntail:].T, ((0, 0), (0, 128 - ntail)))
    stg_u, stg_i = gather(W_user.T, W_item.T, tail_u, tail_i, user, item)

    dot = pl.kernel(
        _dot_body,
        out_type=jax.ShapeDtypeStruct((_BATCH,), jnp.float32),
        mesh=mesh,
        compiler_params=params,
        scratch_types=[
            pltpu.VMEM((256, 128), jnp.float32),
            pltpu.VMEM((256, 128), jnp.float32),
            pltpu.VMEM((_BPW,), jnp.float32),
            pltpu.SemaphoreType.DMA,
        ],
    )
    return dot(stg_u, stg_i)


# speedup vs baseline: 1.6962x; 1.6962x over previous
import jax
import jax.numpy as jnp
from jax import lax
from jax.experimental import pallas as pl
from jax.experimental.pallas import tpu as pltpu
from jax.experimental.pallas import tpu_sc as plsc

_FACTOR = 64
_BATCH = 16384
_ROWS = 1000000
_NC = 2
_NS = 16
_L = 16
_NW = _NC * _NS
_BPW = _BATCH // _NW
_CW = 512
_NCHUNK = (_ROWS + _CW - 1) // _CW
_KMAX = (_NCHUNK + _NW - 1) // _NW


def _iota16():
    return lax.iota(jnp.int32, _L)


def _gather_body(wu_hbm, wi_hbm, tu_hbm, ti_hbm, user_hbm, item_hbm,
                 stg_u, stg_i,
                 idxbuf, lp, lp2, chunk0, chunk1, rowbuf, bidx, sem0, sem1):
    wid = lax.axis_index("s") * _NC + lax.axis_index("c")
    lane = _iota16()
    neg1 = jnp.full((_L,), -1, jnp.int32)

    for wt_hbm, tail_hbm, ix_hbm, stg in ((wu_hbm, tu_hbm, user_hbm, stg_u),
                                          (wi_hbm, ti_hbm, item_hbm, stg_i)):
        def scan_blk(blk, cnt):
            pltpu.sync_copy(ix_hbm.at[pl.ds(blk * 16, 16)], idxbuf)

            def scan4(v4, cnt):
                for u in range(4):
                    v = v4 * 4 + u
                    iv = idxbuf[v >> 3, pl.ds((v & 7) * _L, _L)]
                    c = lax.shift_right_logical(iv, 9)
                    own = (c & (_NW - 1)) == wid
                    csum = plsc.cumsum(jnp.where(own, 1, 0))
                    pos = cnt + csum - 1
                    bvec = blk * 2048 + v * _L + lane
                    packed = (
                        lax.shift_left(lax.shift_right_logical(c, 5), 23)
                        | lax.shift_left(iv & (_CW - 1), 14)
                        | bvec)
                    plsc.store_scatter(lp, [pos], packed, mask=own)
                    cnt = cnt + csum[_L - 1]
                return cnt

            return lax.fori_loop(0, 32, scan4, cnt)

        cnt = lax.fori_loop(0, 8, scan_blk, jnp.int32(0))
        nv = lax.shift_right_logical(cnt + _L - 1, 4)

        gs = []
        ge = []
        gpos = jnp.int32(0)
        for g in range(8):
            gs.append(gpos)

            def regroup_vec(v, m, g=g):
                pv = lp[pl.ds(v * _L, _L)]
                inr = (v * _L + lane) < cnt
                sel = (lax.shift_right_logical(pv, 26) == g) & inr
                csum = plsc.cumsum(jnp.where(sel, 1, 0))
                plsc.store_scatter(lp2, [m + csum - 1], pv, mask=sel)
                return m + csum[_L - 1]

            gpos = lax.fori_loop(0, nv, regroup_vec, gpos)
            ge.append(gpos)

        def clr(v, _):
            bidx[pl.ds(v * _L, _L)] = neg1
            return 0

        lax.fori_loop(0, 8, clr, 0)

        def fire(c, buf, sem):
            @pl.when(c < _NCHUNK - 1)
            def _():
                pltpu.async_copy(
                    wt_hbm.at[:, pl.ds(pl.multiple_of(c * _CW, _CW), _CW)],
                    buf, sem)

            @pl.when(c == _NCHUNK - 1)
            def _():
                pltpu.async_copy(tail_hbm, buf.at[:, pl.ds(0, 128)], sem)

        def wait(c, buf, sem):
            @pl.when(c < _NCHUNK - 1)
            def _():
                pltpu.make_async_copy(
                    wt_hbm.at[:, pl.ds(0, _CW)], buf, sem).wait()

            @pl.when(c == _NCHUNK - 1)
            def _():
                pltpu.make_async_copy(
                    tail_hbm, buf.at[:, pl.ds(0, 128)], sem).wait()

        def flush(mv):
            pltpu.sync_copy(rowbuf,
                            stg.at[plsc.Indices(bidx, ignored_value=-1)])
            lax.fori_loop(0, 8, clr, 0)

        def process(s, buf):
            g = lax.shift_right_logical(s, 3)
            start = gs[7]
            end = ge[7]
            for gg in range(6, -1, -1):
                start = jnp.where(g == gg, gs[gg], start)
                end = jnp.where(g == gg, ge[gg], end)

            def pvec(v, mv):
                pv = lp2[pl.ds(v * _L, _L)]
                posv = v * _L + lane
                inr = (posv >= start) & (posv < end)
                sel = (lax.shift_right_logical(pv, 23) == s) & inr
                npos = plsc.all_reduce_population_count(sel)
                hit = npos[0] > 0

                @pl.when(hit)
                def _():
                    x = lax.shift_right_logical(pv, 14) & (_CW - 1)
                    bv = pv & (_BATCH - 1)
                    row0 = (mv & 7) * _L
                    rowv = row0 + lane
                    fvec = jnp.zeros((_L,), jnp.int32)
                    for _f in range(_FACTOR):
                        vals = plsc.load_gather(buf, [fvec, x])
                        plsc.store_scatter(rowbuf, [rowv, fvec], vals)
                        fvec = fvec + 1
                    bidx[pl.ds(row0, _L)] = jnp.where(sel, bv, -1)

                mv2 = jnp.where(hit, mv + 1, mv)

                @pl.when(hit & ((mv2 & 7) == 0))
                def _():
                    flush(mv2)

                return mv2

            v0 = lax.shift_right_logical(start, 4)
            v1 = lax.shift_right_logical(end + _L - 1, 4)
            mv = lax.fori_loop(v0, v1, pvec, jnp.int32(0))

            @pl.when((mv & 7) != 0)
            def _():
                flush(mv)

        bufs = (chunk0, chunk1)
        sems = (sem0, sem1)
        fire(wid, chunk0, sem0)
        lim = jnp.int32(_NCHUNK)

        def pair(k, _):
            for par in range(2):
                s = k * 2 + par
                c = wid + s * _NW
                cn = wid + (s + 1) * _NW

                @pl.when(cn < lim)
                def _():
                    fire(cn, bufs[1 - par], sems[1 - par])

                @pl.when(c < lim)
                def _():
                    wait(c, bufs[par], sems[par])
                    process(jnp.int32(s), bufs[par])
            return 0

        lax.fori_loop(0, _KMAX // 2, pair, 0)


def _dot_body(stg_u, stg_i, out_hbm, su, si, out_v, sem):
    wid = lax.axis_index("s") * _NC + lax.axis_index("c")
    lane = _iota16()

    for half in range(2):
        row0 = wid * _BPW + half * 256
        pltpu.sync_copy(stg_u.at[pl.ds(row0, 256)], su)
        pltpu.sync_copy(stg_i.at[pl.ds(row0, 256)], si)

        def group(g, _):
            bvec = g * _L + lane
            acc = jnp.zeros((_L,), jnp.float32)
            fvec = jnp.zeros((_L,), jnp.int32)
            for _f in range(_FACTOR):
                u = plsc.load_gather(su, [bvec, fvec])
                v = plsc.load_gather(si, [bvec, fvec])
                acc = acc + u * v
                fvec = fvec + 1
            out_v[pl.ds(half * 256 + g * _L, _L)] = acc
            return 0

        lax.fori_loop(0, 256 // _L, group, 0)

    pltpu.sync_copy(out_v, out_hbm.at[pl.ds(wid * _BPW, _BPW)])


def kernel(user, item, W_user, W_item):
    user = user.astype(jnp.int32).reshape(128, 128)
    item = item.astype(jnp.int32).reshape(128, 128)
    mesh = plsc.VectorSubcoreMesh(core_axis_name="c", subcore_axis_name="s")
    params = pltpu.CompilerParams(needs_layout_passes=False)

    gather = pl.kernel(
        _gather_body,
        out_type=(
            jax.ShapeDtypeStruct((_BATCH, 128), jnp.float32),
            jax.ShapeDtypeStruct((_BATCH, 128), jnp.float32),
        ),
        mesh=mesh,
        compiler_params=params,
        scratch_types=[
            pltpu.VMEM((16, 128), jnp.int32),
            pltpu.VMEM((_BATCH,), jnp.int32),
            pltpu.VMEM((_BATCH,), jnp.int32),
            pltpu.VMEM((_FACTOR, _CW), jnp.float32),
            pltpu.VMEM((_FACTOR, _CW), jnp.float32),
            pltpu.VMEM((128, 128), jnp.float32),
            pltpu.VMEM((128,), jnp.int32),
            pltpu.SemaphoreType.DMA,
            pltpu.SemaphoreType.DMA,
        ],
    )
    ntail = _ROWS - (_NCHUNK - 1) * _CW
    tail_u = jnp.pad(W_user[_ROWS - ntail:].T, ((0, 0), (0, 128 - ntail)))
    tail_i = jnp.pad(W_item[_ROWS - ntail:].T, ((0, 0), (0, 128 - ntail)))
    stg_u, stg_i = gather(W_user.T, W_item.T, tail_u, tail_i, user, item)

    dot = pl.kernel(
        _dot_body,
        out_type=jax.ShapeDtypeStruct((_BATCH,), jnp.float32),
        mesh=mesh,
        compiler_params=params,
        scratch_types=[
            pltpu.VMEM((256, 128), jnp.float32),
            pltpu.VMEM((256, 128), jnp.float32),
            pltpu.VMEM((_BPW,), jnp.float32),
            pltpu.SemaphoreType.DMA,
        ],
    )
    return dot(stg_u, stg_i)


# speedup vs baseline: 1.9616x; 1.1565x over previous
import jax
import jax.numpy as jnp
from jax import lax
from jax.experimental import pallas as pl
from jax.experimental.pallas import tpu as pltpu
from jax.experimental.pallas import tpu_sc as plsc

_FACTOR = 64
_BATCH = 16384
_ROWS = 1000000
_NC = 2
_NS = 16
_L = 16
_NW = _NC * _NS
_BPW = _BATCH // _NW
_CW = 512
_NCHUNK = (_ROWS + _CW - 1) // _CW
_KMAX = (_NCHUNK + _NW - 1) // _NW


def _iota16():
    return lax.iota(jnp.int32, _L)


def _gather_body(wu_hbm, wi_hbm, tu_hbm, ti_hbm, user_hbm, item_hbm,
                 stg_u, stg_i,
                 idxbuf, lp, lp2, chunk0, chunk1, rowbuf, bidx, sem0, sem1):
    wid = lax.axis_index("s") * _NC + lax.axis_index("c")
    lane = _iota16()
    neg1 = jnp.full((_L,), -1, jnp.int32)

    for wt_hbm, tail_hbm, ix_hbm, stg in ((wu_hbm, tu_hbm, user_hbm, stg_u),
                                          (wi_hbm, ti_hbm, item_hbm, stg_i)):
        def scan_blk(blk, cnt):
            pltpu.sync_copy(ix_hbm.at[pl.ds(blk * 16, 16)], idxbuf)

            def scan4(v4, cnt):
                for u in range(4):
                    v = v4 * 4 + u
                    iv = idxbuf[v >> 3, pl.ds((v & 7) * _L, _L)]
                    c = lax.shift_right_logical(iv, 9)
                    own = (c & (_NW - 1)) == wid
                    npos = plsc.all_reduce_population_count(own)

                    @pl.when(npos[0] > 0)
                    def _():
                        csum = plsc.cumsum(jnp.where(own, 1, 0))
                        pos = cnt + csum - 1
                        bvec = blk * 2048 + v * _L + lane
                        packed = (
                            lax.shift_left(lax.shift_right_logical(c, 5), 23)
                            | lax.shift_left(iv & (_CW - 1), 14)
                            | bvec)
                        plsc.store_scatter(lp, [pos], packed, mask=own)

                    cnt = cnt + npos[0]
                return cnt

            return lax.fori_loop(0, 32, scan4, cnt)

        cnt = lax.fori_loop(0, 8, scan_blk, jnp.int32(0))
        nv = lax.shift_right_logical(cnt + _L - 1, 4)

        gs = []
        ge = []
        gpos = jnp.int32(0)
        for g in range(_KMAX):
            gs.append(gpos)

            def regroup_vec(v, m, g=g):
                pv = lp[pl.ds(v * _L, _L)]
                inr = (v * _L + lane) < cnt
                sel = (lax.shift_right_logical(pv, 23) == g) & inr
                npos = plsc.all_reduce_population_count(sel)

                @pl.when(npos[0] > 0)
                def _():
                    csum = plsc.cumsum(jnp.where(sel, 1, 0))
                    plsc.store_scatter(lp2, [m + csum - 1], pv, mask=sel)

                return m + npos[0]

            gpos = lax.fori_loop(0, nv, regroup_vec, gpos)
            ge.append(gpos)

        def clr(v, _):
            bidx[pl.ds(v * _L, _L)] = neg1
            return 0

        lax.fori_loop(0, 8, clr, 0)

        def fire(c, buf, sem):
            @pl.when(c < _NCHUNK - 1)
            def _():
                pltpu.async_copy(
                    wt_hbm.at[:, pl.ds(pl.multiple_of(c * _CW, _CW), _CW)],
                    buf, sem)

            @pl.when(c == _NCHUNK - 1)
            def _():
                pltpu.async_copy(tail_hbm, buf.at[:, pl.ds(0, 128)], sem)

        def wait(c, buf, sem):
            @pl.when(c < _NCHUNK - 1)
            def _():
                pltpu.make_async_copy(
                    wt_hbm.at[:, pl.ds(0, _CW)], buf, sem).wait()

            @pl.when(c == _NCHUNK - 1)
            def _():
                pltpu.make_async_copy(
                    tail_hbm, buf.at[:, pl.ds(0, 128)], sem).wait()

        def flush(mv):
            pltpu.sync_copy(rowbuf,
                            stg.at[plsc.Indices(bidx, ignored_value=-1)])
            lax.fori_loop(0, 8, clr, 0)

        def process(s, buf):
            start = gs[_KMAX - 1]
            end = ge[_KMAX - 1]
            for gg in range(_KMAX - 2, -1, -1):
                start = jnp.where(s == gg, gs[gg], start)
                end = jnp.where(s == gg, ge[gg], end)

            def pvec(v, mv):
                pv = lp2[pl.ds(v * _L, _L)]
                posv = v * _L + lane
                sel = (posv >= start) & (posv < end)
                x = lax.shift_right_logical(pv, 14) & (_CW - 1)
                bv = pv & (_BATCH - 1)
                row0 = (mv & 7) * _L
                rowv = row0 + lane
                fvec = jnp.zeros((_L,), jnp.int32)
                for _f in range(_FACTOR):
                    vals = plsc.load_gather(buf, [fvec, x])
                    plsc.store_scatter(rowbuf, [rowv, fvec], vals)
                    fvec = fvec + 1
                bidx[pl.ds(row0, _L)] = jnp.where(sel, bv, -1)
                mv2 = mv + 1

                @pl.when((mv2 & 7) == 0)
                def _():
                    flush(mv2)

                return mv2

            v0 = lax.shift_right_logical(start, 4)
            v1 = lax.shift_right_logical(end + _L - 1, 4)
            mv = lax.fori_loop(v0, v1, pvec, jnp.int32(0))

            @pl.when((mv & 7) != 0)
            def _():
                flush(mv)

        bufs = (chunk0, chunk1)
        sems = (sem0, sem1)
        fire(wid, chunk0, sem0)
        lim = jnp.int32(_NCHUNK)

        def pair(k, _):
            for par in range(2):
                s = k * 2 + par
                c = wid + s * _NW
                cn = wid + (s + 1) * _NW

                @pl.when(cn < lim)
                def _():
                    fire(cn, bufs[1 - par], sems[1 - par])

                @pl.when(c < lim)
                def _():
                    wait(c, bufs[par], sems[par])
                    process(jnp.int32(s), bufs[par])
            return 0

        lax.fori_loop(0, _KMAX // 2, pair, 0)


def _dot_body(stg_u, stg_i, out_hbm, su, si, out_v, sem):
    wid = lax.axis_index("s") * _NC + lax.axis_index("c")
    lane = _iota16()

    for half in range(2):
        row0 = wid * _BPW + half * 256
        pltpu.sync_copy(stg_u.at[pl.ds(row0, 256)], su)
        pltpu.sync_copy(stg_i.at[pl.ds(row0, 256)], si)

        def group(g, _):
            bvec = g * _L + lane
            acc = jnp.zeros((_L,), jnp.float32)
            fvec = jnp.zeros((_L,), jnp.int32)
            for _f in range(_FACTOR):
                u = plsc.load_gather(su, [bvec, fvec])
                v = plsc.load_gather(si, [bvec, fvec])
                acc = acc + u * v
                fvec = fvec + 1
            out_v[pl.ds(half * 256 + g * _L, _L)] = acc
            return 0

        lax.fori_loop(0, 256 // _L, group, 0)

    pltpu.sync_copy(out_v, out_hbm.at[pl.ds(wid * _BPW, _BPW)])


def kernel(user, item, W_user, W_item):
    user = user.astype(jnp.int32).reshape(128, 128)
    item = item.astype(jnp.int32).reshape(128, 128)
    mesh = plsc.VectorSubcoreMesh(core_axis_name="c", subcore_axis_name="s")
    params = pltpu.CompilerParams(needs_layout_passes=False)

    gather = pl.kernel(
        _gather_body,
        out_type=(
            jax.ShapeDtypeStruct((_BATCH, 128), jnp.float32),
            jax.ShapeDtypeStruct((_BATCH, 128), jnp.float32),
        ),
        mesh=mesh,
        compiler_params=params,
        scratch_types=[
            pltpu.VMEM((16, 128), jnp.int32),
            pltpu.VMEM((_BATCH,), jnp.int32),
            pltpu.VMEM((_BATCH,), jnp.int32),
            pltpu.VMEM((_FACTOR, _CW), jnp.float32),
            pltpu.VMEM((_FACTOR, _CW), jnp.float32),
            pltpu.VMEM((128, 128), jnp.float32),
            pltpu.VMEM((128,), jnp.int32),
            pltpu.SemaphoreType.DMA,
            pltpu.SemaphoreType.DMA,
        ],
    )
    ntail = _ROWS - (_NCHUNK - 1) * _CW
    tail_u = jnp.pad(W_user[_ROWS - ntail:].T, ((0, 0), (0, 128 - ntail)))
    tail_i = jnp.pad(W_item[_ROWS - ntail:].T, ((0, 0), (0, 128 - ntail)))
    stg_u, stg_i = gather(W_user.T, W_item.T, tail_u, tail_i, user, item)

    dot = pl.kernel(
        _dot_body,
        out_type=jax.ShapeDtypeStruct((_BATCH,), jnp.float32),
        mesh=mesh,
        compiler_params=params,
        scratch_types=[
            pltpu.VMEM((256, 128), jnp.float32),
            pltpu.VMEM((256, 128), jnp.float32),
            pltpu.VMEM((_BPW,), jnp.float32),
            pltpu.SemaphoreType.DMA,
        ],
    )
    return dot(stg_u, stg_i)


# speedup vs baseline: 2.2882x; 1.1665x over previous
import jax
import jax.numpy as jnp
from jax import lax
from jax.experimental import pallas as pl
from jax.experimental.pallas import tpu as pltpu
from jax.experimental.pallas import tpu_sc as plsc

_FACTOR = 64
_BATCH = 16384
_ROWS = 1000000
_NC = 2
_NS = 16
_L = 16
_NW = _NC * _NS
_BPW = _BATCH // _NW
_CW = 512
_NCHUNK = (_ROWS + _CW - 1) // _CW
_KMAX = (_NCHUNK + _NW - 1) // _NW


def _iota16():
    return lax.iota(jnp.int32, _L)


def _gather_body(wu_hbm, wi_hbm, tu_hbm, ti_hbm, user_hbm, item_hbm,
                 stg_u, stg_i,
                 idxbuf, lp, lp2, chunk0, chunk1, rowbuf, bidx,
                 sem0, sem1, sem2):
    wid = lax.axis_index("s") * _NC + lax.axis_index("c")
    lane = _iota16()
    neg1 = jnp.full((_L,), -1, jnp.int32)

    for wt_hbm, tail_hbm, ix_hbm, stg in ((wu_hbm, tu_hbm, user_hbm, stg_u),
                                          (wi_hbm, ti_hbm, item_hbm, stg_i)):
        def scan_blk(blk, cnt):
            pltpu.sync_copy(ix_hbm.at[pl.ds(blk * 16, 16)], idxbuf)

            def scan4(v4, cnt):
                for u in range(4):
                    v = v4 * 4 + u
                    iv = idxbuf[v >> 3, pl.ds((v & 7) * _L, _L)]
                    c = lax.shift_right_logical(iv, 9)
                    own = (c & (_NW - 1)) == wid
                    npos = plsc.all_reduce_population_count(own)

                    @pl.when(npos[0] > 0)
                    def _():
                        csum = plsc.cumsum(jnp.where(own, 1, 0))
                        pos = cnt + csum - 1
                        bvec = blk * 2048 + v * _L + lane
                        packed = (
                            lax.shift_left(lax.shift_right_logical(c, 5), 23)
                            | lax.shift_left(iv & (_CW - 1), 14)
                            | bvec)
                        plsc.store_scatter(lp, [pos], packed, mask=own)

                    cnt = cnt + npos[0]
                return cnt

            return lax.fori_loop(0, 32, scan4, cnt)

        cnt = lax.fori_loop(0, 8, scan_blk, jnp.int32(0))
        nv = lax.shift_right_logical(cnt + _L - 1, 4)

        g8s = []
        g8e = []
        gpos = jnp.int32(0)
        for G in range(8):
            g8s.append(gpos)

            def p1(v, m, G=G):
                pv = lp[pl.ds(v * _L, _L)]
                inr = (v * _L + lane) < cnt
                sel = (lax.shift_right_logical(pv, 26) == G) & inr
                npos = plsc.all_reduce_population_count(sel)

                @pl.when(npos[0] > 0)
                def _():
                    csum = plsc.cumsum(jnp.where(sel, 1, 0))
                    plsc.store_scatter(lp2, [m + csum - 1], pv, mask=sel)

                return m + npos[0]

            gpos = lax.fori_loop(0, nv, p1, gpos)
            g8e.append(gpos)

        gs = []
        ge = []
        kpos = jnp.int32(0)
        for k in range(_KMAX):
            G = k >> 3
            gs.append(kpos)

            def p2(v, m, k=k, G=G):
                pv = lp2[pl.ds(v * _L, _L)]
                posv = v * _L + lane
                inr = (posv >= g8s[G]) & (posv < g8e[G])
                sel = (lax.shift_right_logical(pv, 23) == k) & inr
                npos = plsc.all_reduce_population_count(sel)

                @pl.when(npos[0] > 0)
                def _():
                    csum = plsc.cumsum(jnp.where(sel, 1, 0))
                    plsc.store_scatter(lp, [m + csum - 1], pv, mask=sel)

                return m + npos[0]

            v0 = lax.shift_right_logical(g8s[G], 4)
            v1 = lax.shift_right_logical(g8e[G] + _L - 1, 4)
            kpos = lax.fori_loop(v0, v1, p2, kpos)
            ge.append(kpos)

        def clr(v, _):
            bidx[pl.ds(v * _L, _L)] = neg1
            return 0

        lax.fori_loop(0, 8, clr, 0)

        def fire(c, buf, sem):
            @pl.when(c < _NCHUNK - 1)
            def _():
                pltpu.async_copy(
                    wt_hbm.at[:, pl.ds(pl.multiple_of(c * _CW, _CW), _CW)],
                    buf, sem)

            @pl.when(c == _NCHUNK - 1)
            def _():
                pltpu.async_copy(tail_hbm, buf.at[:, pl.ds(0, 128)], sem)

        def wait(c, buf, sem):
            @pl.when(c < _NCHUNK - 1)
            def _():
                pltpu.make_async_copy(
                    wt_hbm.at[:, pl.ds(0, _CW)], buf, sem).wait()

            @pl.when(c == _NCHUNK - 1)
            def _():
                pltpu.make_async_copy(
                    tail_hbm, buf.at[:, pl.ds(0, 128)], sem).wait()

        def flush(mv):
            pltpu.sync_copy(rowbuf,
                            stg.at[plsc.Indices(bidx, ignored_value=-1)])
            lax.fori_loop(0, 8, clr, 0)

        def process(s, buf):
            start = gs[_KMAX - 1]
            end = ge[_KMAX - 1]
            for gg in range(_KMAX - 2, -1, -1):
                start = jnp.where(s == gg, gs[gg], start)
                end = jnp.where(s == gg, ge[gg], end)

            def pvec(v, mv):
                pv = lp[pl.ds(v * _L, _L)]
                posv = v * _L + lane
                sel = (posv >= start) & (posv < end)
                x = lax.shift_right_logical(pv, 14) & (_CW - 1)
                bv = pv & (_BATCH - 1)
                row0 = (mv & 7) * _L
                rowv = row0 + lane
                fvec = jnp.zeros((_L,), jnp.int32)
                for _f in range(_FACTOR):
                    vals = plsc.load_gather(buf, [fvec, x])
                    plsc.store_scatter(rowbuf, [rowv, fvec], vals)
                    fvec = fvec + 1
                bidx[pl.ds(row0, _L)] = jnp.where(sel, bv, -1)
                mv2 = mv + 1

                @pl.when((mv2 & 7) == 0)
                def _():
                    flush(mv2)

                return mv2

            v0 = lax.shift_right_logical(start, 4)
            v1 = lax.shift_right_logical(end + _L - 1, 4)
            mv = lax.fori_loop(v0, v1, pvec, jnp.int32(0))

            @pl.when((mv & 7) != 0)
            def _():
                flush(mv)

        bufs = (chunk0, chunk1)
        sems = (sem0, sem1)
        fire(wid, chunk0, sem0)
        lim = jnp.int32(_NCHUNK)

        def pair(k, _):
            for par in range(2):
                s = k * 2 + par
                c = wid + s * _NW
                cn = wid + (s + 1) * _NW

                @pl.when(cn < lim)
                def _():
                    fire(cn, bufs[1 - par], sems[1 - par])

                @pl.when(c < lim)
                def _():
                    wait(c, bufs[par], sems[par])
                    process(jnp.int32(s), bufs[par])
            return 0

        lax.fori_loop(0, _KMAX // 2, pair, 0)


def _dot_body(stg_u, stg_i, out_hbm, su, si, out_v, sem):
    wid = lax.axis_index("s") * _NC + lax.axis_index("c")
    lane = _iota16()

    for half in range(2):
        row0 = wid * _BPW + half * 256
        pltpu.sync_copy(stg_u.at[pl.ds(row0, 256)], su)
        pltpu.sync_copy(stg_i.at[pl.ds(row0, 256)], si)

        def group(g, _):
            bvec = g * _L + lane
            acc = jnp.zeros((_L,), jnp.float32)
            fvec = jnp.zeros((_L,), jnp.int32)
            for _f in range(_FACTOR):
                u = plsc.load_gather(su, [bvec, fvec])
                v = plsc.load_gather(si, [bvec, fvec])
                acc = acc + u * v
                fvec = fvec + 1
            out_v[pl.ds(half * 256 + g * _L, _L)] = acc
            return 0

        lax.fori_loop(0, 256 // _L, group, 0)

    pltpu.sync_copy(out_v, out_hbm.at[pl.ds(wid * _BPW, _BPW)])


def kernel(user, item, W_user, W_item):
    user = user.astype(jnp.int32).reshape(128, 128)
    item = item.astype(jnp.int32).reshape(128, 128)
    mesh = plsc.VectorSubcoreMesh(core_axis_name="c", subcore_axis_name="s")
    params = pltpu.CompilerParams(needs_layout_passes=False)

    gather = pl.kernel(
        _gather_body,
        out_type=(
            jax.ShapeDtypeStruct((_BATCH, 128), jnp.float32),
            jax.ShapeDtypeStruct((_BATCH, 128), jnp.float32),
        ),
        mesh=mesh,
        compiler_params=params,
        scratch_types=[
            pltpu.VMEM((16, 128), jnp.int32),
            pltpu.VMEM((_BATCH,), jnp.int32),
            pltpu.VMEM((_BATCH,), jnp.int32),
            pltpu.VMEM((_FACTOR, _CW), jnp.float32),
            pltpu.VMEM((_FACTOR, _CW), jnp.float32),
            pltpu.VMEM((128, 128), jnp.float32),
            pltpu.VMEM((128,), jnp.int32),
            pltpu.SemaphoreType.DMA,
            pltpu.SemaphoreType.DMA,
            pltpu.SemaphoreType.DMA,
        ],
    )
    ntail = _ROWS - (_NCHUNK - 1) * _CW
    tail_u = jnp.pad(W_user[_ROWS - ntail:].T, ((0, 0), (0, 128 - ntail)))
    tail_i = jnp.pad(W_item[_ROWS - ntail:].T, ((0, 0), (0, 128 - ntail)))
    stg_u, stg_i = gather(W_user.T, W_item.T, tail_u, tail_i, user, item)

    dot = pl.kernel(
        _dot_body,
        out_type=jax.ShapeDtypeStruct((_BATCH,), jnp.float32),
        mesh=mesh,
        compiler_params=params,
        scratch_types=[
            pltpu.VMEM((256, 128), jnp.float32),
            pltpu.VMEM((256, 128), jnp.float32),
            pltpu.VMEM((_BPW,), jnp.float32),
            pltpu.SemaphoreType.DMA,
        ],
    )
    return dot(stg_u, stg_i)


# speedup vs baseline: 2.4286x; 1.0613x over previous
import jax
import jax.numpy as jnp
from jax import lax
from jax.experimental import pallas as pl
from jax.experimental.pallas import tpu as pltpu
from jax.experimental.pallas import tpu_sc as plsc

_FACTOR = 64
_BATCH = 16384
_ROWS = 1000000
_NC = 2
_NS = 16
_L = 16
_NW = _NC * _NS
_BPW = _BATCH // _NW
_CW = 512
_NCHUNK = (_ROWS + _CW - 1) // _CW
_KMAX = (_NCHUNK + _NW - 1) // _NW


def _iota16():
    return lax.iota(jnp.int32, _L)


def _gather_body(wu_hbm, wi_hbm, tu_hbm, ti_hbm, user_hbm, item_hbm,
                 stg_u, stg_i,
                 idxbuf, lp, lp2, chunk0, chunk1, rowbuf, bidx,
                 sem0, sem1, sem2):
    wid = lax.axis_index("s") * _NC + lax.axis_index("c")
    lane = _iota16()
    neg1 = jnp.full((_L,), -1, jnp.int32)

    for wt_hbm, tail_hbm, ix_hbm, stg in ((wu_hbm, tu_hbm, user_hbm, stg_u),
                                          (wi_hbm, ti_hbm, item_hbm, stg_i)):
        def scan_blk(blk, cnt):
            pltpu.sync_copy(ix_hbm.at[pl.ds(blk * 16, 16)], idxbuf)

            def scan4(v4, cnt):
                for u in range(4):
                    v = v4 * 4 + u
                    iv = idxbuf[v >> 3, pl.ds((v & 7) * _L, _L)]
                    c = lax.shift_right_logical(iv, 9)
                    own = (c & (_NW - 1)) == wid
                    npos = plsc.all_reduce_population_count(own)

                    @pl.when(npos[0] > 0)
                    def _():
                        csum = plsc.cumsum(jnp.where(own, 1, 0))
                        pos = cnt + csum - 1
                        bvec = blk * 2048 + v * _L + lane
                        packed = (
                            lax.shift_left(lax.shift_right_logical(c, 5), 23)
                            | lax.shift_left(iv & (_CW - 1), 14)
                            | bvec)
                        plsc.store_scatter(lp, [pos], packed, mask=own)

                    cnt = cnt + npos[0]
                return cnt

            return lax.fori_loop(0, 32, scan4, cnt)

        cnt = lax.fori_loop(0, 8, scan_blk, jnp.int32(0))
        nv = lax.shift_right_logical(cnt + _L - 1, 4)

        g8s = []
        g8e = []
        gpos = jnp.int32(0)
        for G in range(8):
            g8s.append(gpos)

            def p1(v, m, G=G):
                pv = lp[pl.ds(v * _L, _L)]
                inr = (v * _L + lane) < cnt
                sel = (lax.shift_right_logical(pv, 26) == G) & inr
                npos = plsc.all_reduce_population_count(sel)

                @pl.when(npos[0] > 0)
                def _():
                    csum = plsc.cumsum(jnp.where(sel, 1, 0))
                    plsc.store_scatter(lp2, [m + csum - 1], pv, mask=sel)

                return m + npos[0]

            gpos = lax.fori_loop(0, nv, p1, gpos)
            g8e.append(gpos)

        gs = []
        ge = []
        kpos = jnp.int32(0)
        for k in range(_KMAX):
            G = k >> 3
            gs.append(kpos)

            def p2(v, m, k=k, G=G):
                pv = lp2[pl.ds(v * _L, _L)]
                posv = v * _L + lane
                inr = (posv >= g8s[G]) & (posv < g8e[G])
                sel = (lax.shift_right_logical(pv, 23) == k) & inr
                npos = plsc.all_reduce_population_count(sel)

                @pl.when(npos[0] > 0)
                def _():
                    csum = plsc.cumsum(jnp.where(sel, 1, 0))
                    plsc.store_scatter(lp, [m + csum - 1], pv, mask=sel)

                return m + npos[0]

            v0 = lax.shift_right_logical(g8s[G], 4)
            v1 = lax.shift_right_logical(g8e[G] + _L - 1, 4)
            kpos = lax.fori_loop(v0, v1, p2, kpos)
            ge.append(kpos)

        def clr(v, _):
            bidx[pl.ds(v * _L, _L)] = neg1
            return 0

        lax.fori_loop(0, 8, clr, 0)

        def fire(c, buf, sem):
            @pl.when(c < _NCHUNK - 1)
            def _():
                pltpu.async_copy(
                    wt_hbm.at[:, pl.ds(pl.multiple_of(c * _CW, _CW), _CW)],
                    buf, sem)

            @pl.when(c == _NCHUNK - 1)
            def _():
                pltpu.async_copy(tail_hbm, buf.at[:, pl.ds(0, 128)], sem)

        def wait(c, buf, sem):
            @pl.when(c < _NCHUNK - 1)
            def _():
                pltpu.make_async_copy(
                    wt_hbm.at[:, pl.ds(0, _CW)], buf, sem).wait()

            @pl.when(c == _NCHUNK - 1)
            def _():
                pltpu.make_async_copy(
                    tail_hbm, buf.at[:, pl.ds(0, 128)], sem).wait()

        def flush(mv):
            pltpu.sync_copy(rowbuf,
                            stg.at[plsc.Indices(bidx, ignored_value=-1)])
            lax.fori_loop(0, 8, clr, 0)

        def process(s, buf):
            start = gs[_KMAX - 1]
            end = ge[_KMAX - 1]
            for gg in range(_KMAX - 2, -1, -1):
                start = jnp.where(s == gg, gs[gg], start)
                end = jnp.where(s == gg, ge[gg], end)

            def pvec(v, mv):
                pv = lp[pl.ds(v * _L, _L)]
                posv = v * _L + lane
                sel = (posv >= start) & (posv < end)
                x = lax.shift_right_logical(pv, 14) & (_CW - 1)
                bv = pv & (_BATCH - 1)
                row0 = (mv & 7) * _L
                rowv = row0 + lane
                fvec = jnp.zeros((_L,), jnp.int32)
                for _f in range(_FACTOR):
                    vals = plsc.load_gather(buf, [fvec, x])
                    plsc.store_scatter(rowbuf, [rowv, fvec], vals)
                    fvec = fvec + 1
                bidx[pl.ds(row0, _L)] = jnp.where(sel, bv, -1)
                mv2 = mv + 1

                @pl.when((mv2 & 7) == 0)
                def _():
                    flush(mv2)

                return mv2

            v0 = lax.shift_right_logical(start, 4)
            v1 = lax.shift_right_logical(end + _L - 1, 4)
            lax.fori_loop(v0, v1, pvec, jnp.int32(0))

        bufs = (chunk0, chunk1)
        sems = (sem0, sem1)
        fire(wid, chunk0, sem0)
        lim = jnp.int32(_NCHUNK)

        def mk_flush():
            return pltpu.make_async_copy(
                rowbuf, stg.at[plsc.Indices(bidx, ignored_value=-1)], sem2)

        def pair(k, _):
            for par in range(2):
                s = k * 2 + par
                c = wid + s * _NW
                cn = wid + (s + 1) * _NW

                @pl.when(cn < lim)
                def _():
                    fire(cn, bufs[1 - par], sems[1 - par])

                @pl.when(jnp.int32(s) > 0)
                def _():
                    mk_flush().wait()

                lax.fori_loop(0, 8, clr, 0)

                @pl.when(c < lim)
                def _():
                    wait(c, bufs[par], sems[par])
                    process(jnp.int32(s), bufs[par])

                mk_flush().start()
            return 0

        lax.fori_loop(0, _KMAX // 2, pair, 0)
        mk_flush().wait()


def _dot_body(stg_u, stg_i, out_hbm, su, si, out_v, sem):
    wid = lax.axis_index("s") * _NC + lax.axis_index("c")
    lane = _iota16()

    for half in range(2):
        row0 = wid * _BPW + half * 256
        pltpu.sync_copy(stg_u.at[pl.ds(row0, 256)], su)
        pltpu.sync_copy(stg_i.at[pl.ds(row0, 256)], si)

        def group(g, _):
            bvec = g * _L + lane
            acc = jnp.zeros((_L,), jnp.float32)
            fvec = jnp.zeros((_L,), jnp.int32)
            for _f in range(_FACTOR):
                u = plsc.load_gather(su, [bvec, fvec])
                v = plsc.load_gather(si, [bvec, fvec])
                acc = acc + u * v
                fvec = fvec + 1
            out_v[pl.ds(half * 256 + g * _L, _L)] = acc
            return 0

        lax.fori_loop(0, 256 // _L, group, 0)

    pltpu.sync_copy(out_v, out_hbm.at[pl.ds(wid * _BPW, _BPW)])


def kernel(user, item, W_user, W_item):
    user = user.astype(jnp.int32).reshape(128, 128)
    item = item.astype(jnp.int32).reshape(128, 128)
    mesh = plsc.VectorSubcoreMesh(core_axis_name="c", subcore_axis_name="s")
    params = pltpu.CompilerParams(needs_layout_passes=False)

    gather = pl.kernel(
        _gather_body,
        out_type=(
            jax.ShapeDtypeStruct((_BATCH, 128), jnp.float32),
            jax.ShapeDtypeStruct((_BATCH, 128), jnp.float32),
        ),
        mesh=mesh,
        compiler_params=params,
        scratch_types=[
            pltpu.VMEM((16, 128), jnp.int32),
            pltpu.VMEM((_BATCH,), jnp.int32),
            pltpu.VMEM((_BATCH,), jnp.int32),
            pltpu.VMEM((_FACTOR, _CW), jnp.float32),
            pltpu.VMEM((_FACTOR, _CW), jnp.float32),
            pltpu.VMEM((128, 128), jnp.float32),
            pltpu.VMEM((128,), jnp.int32),
            pltpu.SemaphoreType.DMA,
            pltpu.SemaphoreType.DMA,
            pltpu.SemaphoreType.DMA,
        ],
    )
    ntail = _ROWS - (_NCHUNK - 1) * _CW
    tail_u = jnp.pad(W_user[_ROWS - ntail:].T, ((0, 0), (0, 128 - ntail)))
    tail_i = jnp.pad(W_item[_ROWS - ntail:].T, ((0, 0), (0, 128 - ntail)))
    stg_u, stg_i = gather(W_user.T, W_item.T, tail_u, tail_i, user, item)

    dot = pl.kernel(
        _dot_body,
        out_type=jax.ShapeDtypeStruct((_BATCH,), jnp.float32),
        mesh=mesh,
        compiler_params=params,
        scratch_types=[
            pltpu.VMEM((256, 128), jnp.float32),
            pltpu.VMEM((256, 128), jnp.float32),
            pltpu.VMEM((_BPW,), jnp.float32),
            pltpu.SemaphoreType.DMA,
        ],
    )
    return dot(stg_u, stg_i)
